# Initial kernel scaffold; baseline (speedup 1.0000x reference)
import jax
import jax.numpy as jnp
from jax.experimental import pallas as pl


def kernel(x, edge_index, vp, batch, W1, b1, g1, be1, W2, b2, g2, be2, W3, b3, g3, be3, W4, b4):
    raise NotImplementedError("write your pallas kernel here")


# speedup vs baseline: 3.4837x; 3.4837x over previous
import functools

import jax
import jax.numpy as jnp
from jax import lax
from jax.experimental import pallas as pl
from jax.experimental.pallas import tpu as pltpu
from jax.experimental.pallas import tpu_sc as plsc

N = 10000
E = 320000
D = 128
H = 8
G = 64
EPS = 1e-5

NC = 2
NS = 16
NW = NC * NS
EPW = E // NW
CHUNK = 2000
NCHUNK = EPW // CHUNK

BM = 512
GRID = E // BM


def _precompute_body(x_ref, vp_ref, batch_ref, W1_ref, b1_ref, T_ref):
    W1ab = jnp.concatenate([W1_ref[0:D, :], W1_ref[D:2 * D, :]], axis=1)
    W1c = W1_ref[2 * D:3 * D, :]
    R = jnp.dot(vp_ref[:], W1c, preferred_element_type=jnp.float32) + b1_ref[:]
    onehot = (batch_ref[:] == lax.broadcasted_iota(jnp.int32, (N, G), 1)
              ).astype(jnp.float32)
    Rb = jnp.dot(onehot, R, preferred_element_type=jnp.float32)
    T = jnp.dot(x_ref[:], W1ab, preferred_element_type=jnp.float32)
    T_ref[:] = T + jnp.concatenate([Rb, jnp.zeros((N, H), jnp.float32)], axis=1)


def _precompute(x, vp, batch2d, W1, b1row):
    return pl.pallas_call(
        _precompute_body,
        out_shape=jax.ShapeDtypeStruct((N, 2 * H), jnp.float32),
    )(x, vp, batch2d, W1, b1row)


def _make_gather():
    mesh = plsc.VectorSubcoreMesh(core_axis_name="c", subcore_axis_name="s")

    @functools.partial(
        pl.kernel,
        mesh=mesh,
        out_type=[jax.ShapeDtypeStruct((E, 2 * H), jnp.float32),
                  jax.ShapeDtypeStruct((E, 2 * H), jnp.float32)],
        scratch_types=[pltpu.VMEM((CHUNK,), jnp.int32),
                       pltpu.VMEM((CHUNK,), jnp.int32),
                       pltpu.VMEM((CHUNK, 2 * H), jnp.float32),
                       pltpu.VMEM((CHUNK, 2 * H), jnp.float32),
                       pltpu.SemaphoreType.DMA,
                       pltpu.SemaphoreType.DMA],
        compiler_params=pltpu.CompilerParams(use_tc_tiling_on_sc=False),
    )
    def gather_k(T_hbm, s_hbm, e_hbm, outS, outE, idxs, idxe, rs, re, sem1, sem2):
        wid = lax.axis_index("s") * NC + lax.axis_index("c")
        base = pl.multiple_of(wid * EPW, 8)
        for c in range(NCHUNK):
            off = pl.multiple_of(base + c * CHUNK, 8)
            pltpu.sync_copy(s_hbm.at[pl.ds(off, CHUNK)], idxs)
            pltpu.sync_copy(e_hbm.at[pl.ds(off, CHUNK)], idxe)
            c1 = pltpu.async_copy(T_hbm.at[idxs], rs, sem1)
            c2 = pltpu.async_copy(T_hbm.at[idxe], re, sem2)
            c1.wait()
            c2.wait()
            pltpu.sync_copy(rs, outS.at[pl.ds(off, CHUNK)])
            pltpu.sync_copy(re, outE.at[pl.ds(off, CHUNK)])

    return gather_k


_gather = _make_gather()


def _ln(h, g, b):
    m = jnp.mean(h, axis=-1, keepdims=True)
    v = jnp.var(h, axis=-1, keepdims=True)
    return (h - m) / jnp.sqrt(v + EPS) * g + b


def _mlp_body(gs_ref, ge_ref, g1_ref, be1_ref, W2_ref, b2_ref, g2_ref,
              be2_ref, W3_ref, b3_ref, g3_ref, be3_ref, w4_ref, b4_ref,
              out_ref):
    z = gs_ref[:, 0:H] + ge_ref[:, H:2 * H]
    h = jnp.tanh(_ln(z, g1_ref[:], be1_ref[:]))
    h = jnp.tanh(_ln(jnp.dot(h, W2_ref[:], preferred_element_type=jnp.float32)
                     + b2_ref[:], g2_ref[:], be2_ref[:]))
    h = jnp.tanh(_ln(jnp.dot(h, W3_ref[:], preferred_element_type=jnp.float32)
                     + b3_ref[:], g3_ref[:], be3_ref[:]))
    out_ref[:] = jnp.sum(h * w4_ref[:], axis=1) + b4_ref[0]


def _mlp(gs, ge, g1, be1, W2, b2, g2, be2, W3, b3, g3, be3, w4row, b4):
    row = pl.BlockSpec((1, H), lambda i: (0, 0))
    mat = pl.BlockSpec((H, H), lambda i: (0, 0))
    return pl.pallas_call(
        _mlp_body,
        grid=(GRID,),
        in_specs=[
            pl.BlockSpec((BM, 2 * H), lambda i: (i, 0)),
            pl.BlockSpec((BM, 2 * H), lambda i: (i, 0)),
            row, row, mat, row, row, row, mat, row, row, row, row,
            pl.BlockSpec(memory_space=pltpu.SMEM),
        ],
        out_specs=pl.BlockSpec((BM,), lambda i: (i,)),
        out_shape=jax.ShapeDtypeStruct((E,), jnp.float32),
    )(gs, ge, g1, be1, W2, b2, g2, be2, W3, b3, g3, be3, w4row, b4)


def kernel(x, edge_index, vp, batch, W1, b1, g1, be1, W2, b2, g2, be2,
           W3, b3, g3, be3, W4, b4):
    T = _precompute(x, vp, batch.reshape(N, 1), W1, b1.reshape(1, H))
    start = edge_index[0]
    end = edge_index[1]
    gs, ge = _gather(T, start, end)
    return _mlp(gs, ge, g1.reshape(1, H), be1.reshape(1, H), W2,
                b2.reshape(1, H), g2.reshape(1, H), be2.reshape(1, H), W3,
                b3.reshape(1, H), g3.reshape(1, H), be3.reshape(1, H),
                W4.reshape(1, H), b4)
